# baseline (speedup 1.0000x reference)
import functools

import numpy as np

import jax
import jax.numpy as jnp
from jax.experimental import pallas as pl
from jax.experimental.pallas import tpu as pltpu

N_SLOT = 112
N_HEAD = 8
DIM = 512
HEAD_DIM = 64
SLOT_PAD = 128
CAT = N_HEAD * SLOT_PAD
RADIUS = 16.0
EPS = 1e-5
TB = 512


def _f32dot(a, b):
    return jnp.dot(a, b, preferred_element_type=jnp.float32)


def _dot_rhs_t(a, b):
    return jax.lax.dot_general(a, b, (((1,), (1,)), ((), ())),
                               preferred_element_type=jnp.float32)


def _rownorm(x):
    ss = jnp.sum(x * x, axis=-1, keepdims=True)
    return x * jax.lax.rsqrt(jnp.maximum(ss, 1e-24))


def _ln_mxu(x, g, b, ones_ref):
    xb = x.astype(jnp.bfloat16)
    m = jnp.dot(xb, ones_ref[...], preferred_element_type=jnp.float32)
    s = jnp.dot(xb * xb, ones_ref[...], preferred_element_type=jnp.float32)
    r = jax.lax.rsqrt(jnp.maximum(s - m * m, 0.0) + EPS)
    return (x - pltpu.repeat(m, 4, axis=1)) * pltpu.repeat(r, 4, axis=1) \
        * g + b


def _prep_kernel(mkt_ref, mvp_ref, mvpt_ref, ow_ref, perm_ref,
                 kpadt_ref, wcat_ref, vnpt_ref, closs_ref,
                 kscr_ref, wscr_ref):
    mkt = mkt_ref[...]
    sst = jnp.sum(mkt * mkt, axis=0, keepdims=True)
    knt = mkt * jax.lax.rsqrt(jnp.maximum(sst, 1e-24))
    mvp = mvp_ref[...]
    for h in range(N_HEAD):
        bh = knt[:, h * N_SLOT:(h + 1) * N_SLOT]
        kscr_ref[h * HEAD_DIM:(h + 1) * HEAD_DIM, :] = jnp.pad(
            bh, ((0, 0), (h * SLOT_PAD, CAT - h * SLOT_PAD - N_SLOT)),
        ).astype(jnp.bfloat16)
        wscr_ref[h * SLOT_PAD:(h + 1) * SLOT_PAD, :] = _dot_rhs_t(
            mvp, ow_ref[:, h * DIM:(h + 1) * DIM]).astype(jnp.bfloat16)
    kpadt_ref[...] = _f32dot(kscr_ref[...], perm_ref[...]).astype(jnp.bfloat16)
    wcat_ref[...] = jax.lax.dot_general(
        perm_ref[...], wscr_ref[...], (((0,), (0,)), ((), ())),
        preferred_element_type=jnp.float32).astype(jnp.bfloat16)
    mvpt = mvpt_ref[...]
    ssv = jnp.sum(mvpt * mvpt, axis=0, keepdims=True)
    vnt = mvpt * jax.lax.rsqrt(jnp.maximum(ssv, 1e-24))
    vnpt_ref[...] = vnt.astype(jnp.bfloat16)
    gram = jax.lax.dot_general(vnt, vnt, (((0,), (0,)), ((), ())),
                               preferred_element_type=jnp.float32)
    r = jax.lax.broadcasted_iota(jnp.int32, (SLOT_PAD, SLOT_PAD), 0)
    c = jax.lax.broadcasted_iota(jnp.int32, (SLOT_PAD, SLOT_PAD), 1)
    eye = jnp.where((r == c) & (r < N_SLOT), 1.0, 0.0)
    closs_ref[...] = jnp.full((8, 128), jnp.sum(jnp.abs(eye - gram)) * 0.01,
                              jnp.float32)


def _main_kernel(q_ref, v_ref, qwt_ref, vwt_ref, kpadt_ref, wcat_ref,
                 vnpt_ref, mvp_ref, sel_ref, seg_ref, ones_ref, onesf_ref,
                 bias_ref, fp_ref, ft_ref, part_ref):
    bf16 = jnp.bfloat16
    LOG2E = 1.4426950408889634
    ln1_g, ln1_b = bias_ref[3:4, :], bias_ref[4:5, :]

    def _half(r0, rows):
        sl = slice(r0, r0 + rows)
        q = q_ref[sl, :]
        v = v_ref[sl, :]

        qp = _f32dot(q.astype(bf16), qwt_ref[...]) + bias_ref[0:1, :]
        qpb = qp.astype(bf16)
        ssh = _f32dot(qpb * qpb, sel_ref[...])
        invc = jax.lax.rsqrt(jnp.maximum(ssh, 1e-24)) * (RADIUS * LOG2E)
        raw = _f32dot(qpb, kpadt_ref[...])
        e = jnp.exp2(raw * pltpu.repeat(invc, N_HEAD, axis=1))
        eb = e.astype(bf16)
        ssum = _f32dot(eb, seg_ref[...]) - 16.0
        rec = (1.0 / jnp.maximum(ssum, 1e-30)).astype(bf16)
        addr = eb * pltpu.repeat(rec, N_HEAD, axis=1)
        attn = _f32dot(addr, wcat_ref[...]) + bias_ref[2:3, :]
        fp_ref[sl, :] = _ln_mxu(q + attn, ln1_g, ln1_b, ones_ref)

        vb = v.astype(bf16)
        vp = _f32dot(vb, vwt_ref[...]) + bias_ref[1:2, :]
        vpb = vp.astype(bf16)
        ssv = _f32dot(vpb * vpb, ones_ref[...]) * DIM
        rc = jax.lax.rsqrt(jnp.maximum(ssv, 1e-24)) * (RADIUS * LOG2E)
        raw2 = _f32dot(vpb, vnpt_ref[...])
        e2 = jnp.exp2(raw2 * rc)
        e2b = e2.astype(bf16)
        z2 = _f32dot(e2b, onesf_ref[...]) - 16.0
        addr2 = e2b * (1.0 / jnp.maximum(z2, 1e-30)).astype(bf16)
        ar = _f32dot(addr2, mvp_ref[...])
        ab = ar.astype(bf16)
        s_av = _f32dot(ab * vb, ones_ref[...])
        s_aa = _f32dot(ab * ab, ones_ref[...])
        s_vv = _f32dot(vb * vb, ones_ref[...])
        cos = s_av * jax.lax.rsqrt(jnp.maximum(s_aa * s_vv, 1e-30))
        ft_ref[sl, :] = _ln_mxu(
            q + _ln_mxu(ar, bias_ref[5:6, :], bias_ref[6:7, :], ones_ref),
            ln1_g, ln1_b, ones_ref)
        return jnp.sum(jnp.abs(1.0 - cos), axis=0, keepdims=True)

    part = _half(0, TB)
    part_ref[...] = part.reshape(1, 1, 128)


def kernel(query, value, mem_key, mem_value, q_w, q_b, v_w, v_b,
           out_w, out_b, ln1_g, ln1_b, ln3_g, ln3_b):
    B, S, C = query.shape
    N = B * S
    G = N // TB
    f32 = jnp.float32
    q2 = query.reshape(N, C)
    v2 = value.reshape(N, DIM)
    mvp = jnp.pad(mem_value, ((0, SLOT_PAD - N_SLOT), (0, 0)))

    co = np.arange(CAT)
    perm_np = np.zeros((CAT, CAT), np.float32)
    perm_np[co, (co % SLOT_PAD) * N_HEAD + co // SLOT_PAD] = 1.0
    perm = jnp.asarray(perm_np, jnp.bfloat16)

    kpadt, wcat, vnpt, closs_arr = pl.pallas_call(
        _prep_kernel,
        out_shape=[
            jax.ShapeDtypeStruct((DIM, CAT), jnp.bfloat16),
            jax.ShapeDtypeStruct((CAT, DIM), jnp.bfloat16),
            jax.ShapeDtypeStruct((DIM, SLOT_PAD), jnp.bfloat16),
            jax.ShapeDtypeStruct((8, 128), f32),
        ],
        scratch_shapes=[
            pltpu.VMEM((DIM, CAT), jnp.bfloat16),
            pltpu.VMEM((CAT, DIM), jnp.bfloat16),
        ],
        name="mem_prep",
    )(mem_key.T, mvp, mvp.T, out_w, perm)

    di = np.arange(DIM)
    hi = np.arange(128)
    ci = np.arange(CAT)
    sel = jnp.asarray((di[:, None] // HEAD_DIM == hi[None, :] % N_HEAD),
                      jnp.bfloat16)
    seg = jnp.asarray((ci[:, None] % N_HEAD == hi[None, :] % N_HEAD),
                      jnp.bfloat16)
    ones = jnp.full((DIM, 128), 1.0 / DIM, jnp.bfloat16)
    onesf = jnp.ones((SLOT_PAD, 128), jnp.bfloat16)
    bias_pack = jnp.stack(
        [q_b, v_b, out_b, ln1_g, ln1_b, ln3_g, ln3_b, jnp.zeros_like(q_b)],
        axis=0)

    res = lambda shape: pl.BlockSpec(shape, lambda i: (0,) * len(shape))
    blk = lambda i: (i, 0)
    fp, ft, parts = pl.pallas_call(
        _main_kernel,
        grid=(G,),
        in_specs=[
            pl.BlockSpec((TB, DIM), blk),
            pl.BlockSpec((TB, DIM), blk),
            res((DIM, DIM)),
            res((DIM, DIM)),
            res((DIM, CAT)),
            res((CAT, DIM)),
            res((DIM, SLOT_PAD)),
            res((SLOT_PAD, DIM)),
            res((DIM, 128)),
            res((CAT, 128)),
            res((DIM, 128)),
            res((SLOT_PAD, 128)),
            res((8, DIM)),
        ],
        out_specs=[
            pl.BlockSpec((TB, DIM), blk),
            pl.BlockSpec((TB, DIM), blk),
            pl.BlockSpec((1, 1, 128), lambda i: (i, 0, 0)),
        ],
        out_shape=[
            jax.ShapeDtypeStruct((N, DIM), f32),
            jax.ShapeDtypeStruct((N, DIM), f32),
            jax.ShapeDtypeStruct((G, 1, 128), f32),
        ],
        compiler_params=pltpu.CompilerParams(
            dimension_semantics=("parallel",),
            vmem_limit_bytes=48 * 1024 * 1024,
        ),
        name="mem_main",
    )(q2, v2, q_w.T.astype(jnp.bfloat16), v_w.T.astype(jnp.bfloat16),
      kpadt, wcat, vnpt, mvp.astype(jnp.bfloat16), sel, seg, ones, onesf,
      bias_pack)

    f_predict = fp.reshape(B, S, C)
    f_target_recon = ft.reshape(B, S, C)
    recon_loss = jnp.sum(parts[:, 0, 0]) / N
    return (f_predict, f_target_recon, recon_loss, closs_arr[0, 0])


# speedup vs baseline: 1.1666x; 1.1666x over previous
import functools

import numpy as np

import jax
import jax.numpy as jnp
from jax.experimental import pallas as pl
from jax.experimental.pallas import tpu as pltpu

N_SLOT = 112
N_HEAD = 8
DIM = 512
HEAD_DIM = 64
SLOT_PAD = 128
CAT = N_HEAD * SLOT_PAD
RADIUS = 16.0
EPS = 1e-5
TB = 512


def _f32dot(a, b):
    return jnp.dot(a, b, preferred_element_type=jnp.float32)


def _dot_rhs_t(a, b):
    return jax.lax.dot_general(a, b, (((1,), (1,)), ((), ())),
                               preferred_element_type=jnp.float32)


def _rownorm(x):
    ss = jnp.sum(x * x, axis=-1, keepdims=True)
    return x * jax.lax.rsqrt(jnp.maximum(ss, 1e-24))


def _ln(x, g, b):
    m = jnp.mean(x, axis=-1, keepdims=True)
    m2 = jnp.mean(x * x, axis=-1, keepdims=True)
    r = jax.lax.rsqrt(jnp.maximum(m2 - m * m, 0.0) + EPS)
    return (x - m) * r * g + b


def _prep_kernel(mkt_ref, mvp_ref, mvpt_ref, ow_ref, perm_ref,
                 kpadt_ref, wcat_ref, vnpt_ref, closs_ref,
                 kscr_ref, wscr_ref):
    mkt = mkt_ref[...]
    sst = jnp.sum(mkt * mkt, axis=0, keepdims=True)
    knt = mkt * jax.lax.rsqrt(jnp.maximum(sst, 1e-24))
    mvp = mvp_ref[...]
    for h in range(N_HEAD):
        bh = knt[:, h * N_SLOT:(h + 1) * N_SLOT]
        kscr_ref[h * HEAD_DIM:(h + 1) * HEAD_DIM, :] = jnp.pad(
            bh, ((0, 0), (h * SLOT_PAD, CAT - h * SLOT_PAD - N_SLOT)),
        ).astype(jnp.bfloat16)
        wscr_ref[h * SLOT_PAD:(h + 1) * SLOT_PAD, :] = _dot_rhs_t(
            mvp, ow_ref[:, h * DIM:(h + 1) * DIM]).astype(jnp.bfloat16)
    kpadt_ref[...] = _f32dot(kscr_ref[...], perm_ref[...]).astype(jnp.bfloat16)
    wcat_ref[...] = jax.lax.dot_general(
        perm_ref[...], wscr_ref[...], (((0,), (0,)), ((), ())),
        preferred_element_type=jnp.float32).astype(jnp.bfloat16)
    mvpt = mvpt_ref[...]
    ssv = jnp.sum(mvpt * mvpt, axis=0, keepdims=True)
    vnt = mvpt * jax.lax.rsqrt(jnp.maximum(ssv, 1e-24))
    vnpt_ref[...] = vnt.astype(jnp.bfloat16)
    gram = jax.lax.dot_general(vnt, vnt, (((0,), (0,)), ((), ())),
                               preferred_element_type=jnp.float32)
    r = jax.lax.broadcasted_iota(jnp.int32, (SLOT_PAD, SLOT_PAD), 0)
    c = jax.lax.broadcasted_iota(jnp.int32, (SLOT_PAD, SLOT_PAD), 1)
    eye = jnp.where((r == c) & (r < N_SLOT), 1.0, 0.0)
    closs_ref[...] = jnp.full((8, 128), jnp.sum(jnp.abs(eye - gram)) * 0.01,
                              jnp.float32)


def _main_kernel(q_ref, v_ref, qwt_ref, vwt_ref, kpadt_ref, wcat_ref,
                 vnpt_ref, mvp_ref, sel_ref, seg_ref,
                 bias_ref, fp_ref, ft_ref, part_ref):
    bf16 = jnp.bfloat16
    LOG2E = 1.4426950408889634
    ln1_g, ln1_b = bias_ref[3:4, :], bias_ref[4:5, :]

    def _half(r0, rows):
        sl = slice(r0, r0 + rows)
        q = q_ref[sl, :]
        v = v_ref[sl, :]

        qp = _f32dot(q.astype(bf16), qwt_ref[...]) + bias_ref[0:1, :]
        qpb = qp.astype(bf16)
        ssh = _f32dot(qpb * qpb, sel_ref[...])
        invc = jax.lax.rsqrt(jnp.maximum(ssh, 1e-24)) * (RADIUS * LOG2E)
        raw = _f32dot(qpb, kpadt_ref[...])
        e = jnp.exp2(raw * pltpu.repeat(invc, N_HEAD, axis=1))
        eb = e.astype(bf16)
        ssum = _f32dot(eb, seg_ref[...]) - 16.0
        rec = (1.0 / jnp.maximum(ssum, 1e-30)).astype(bf16)
        addr = eb * pltpu.repeat(rec, N_HEAD, axis=1)
        attn = _f32dot(addr, wcat_ref[...]) + bias_ref[2:3, :]
        fp_ref[sl, :] = _ln(q + attn, ln1_g, ln1_b)

        vp = _f32dot(v.astype(bf16), vwt_ref[...]) + bias_ref[1:2, :]
        ssv = jnp.sum(vp * vp, axis=-1, keepdims=True)
        rc = jax.lax.rsqrt(jnp.maximum(ssv, 1e-24)) * (RADIUS * LOG2E)
        raw2 = _f32dot(vp.astype(bf16), vnpt_ref[...])
        e2 = jnp.exp2(raw2 * rc)
        addr2 = e2 / (jnp.sum(e2, axis=-1, keepdims=True) - 16.0)
        ar = _f32dot(addr2.astype(bf16), mvp_ref[...])
        s_av = jnp.sum(ar * v, axis=-1, keepdims=True)
        s_aa = jnp.sum(ar * ar, axis=-1, keepdims=True)
        s_vv = jnp.sum(v * v, axis=-1, keepdims=True)
        cos = s_av * jax.lax.rsqrt(jnp.maximum(s_aa * s_vv, 1e-24))
        ft_ref[sl, :] = _ln(q + _ln(ar, bias_ref[5:6, :], bias_ref[6:7, :]),
                            ln1_g, ln1_b)
        return jnp.sum(jnp.abs(1.0 - cos))

    part = _half(0, TB)
    part_ref[...] = jnp.full((1, 1, 128), part, jnp.float32)


def kernel(query, value, mem_key, mem_value, q_w, q_b, v_w, v_b,
           out_w, out_b, ln1_g, ln1_b, ln3_g, ln3_b):
    B, S, C = query.shape
    N = B * S
    G = N // TB
    f32 = jnp.float32
    q2 = query.reshape(N, C)
    v2 = value.reshape(N, DIM)
    mvp = jnp.pad(mem_value, ((0, SLOT_PAD - N_SLOT), (0, 0)))

    co = np.arange(CAT)
    perm_np = np.zeros((CAT, CAT), np.float32)
    perm_np[co, (co % SLOT_PAD) * N_HEAD + co // SLOT_PAD] = 1.0
    perm = jnp.asarray(perm_np, jnp.bfloat16)

    kpadt, wcat, vnpt, closs_arr = pl.pallas_call(
        _prep_kernel,
        out_shape=[
            jax.ShapeDtypeStruct((DIM, CAT), jnp.bfloat16),
            jax.ShapeDtypeStruct((CAT, DIM), jnp.bfloat16),
            jax.ShapeDtypeStruct((DIM, SLOT_PAD), jnp.bfloat16),
            jax.ShapeDtypeStruct((8, 128), f32),
        ],
        scratch_shapes=[
            pltpu.VMEM((DIM, CAT), jnp.bfloat16),
            pltpu.VMEM((CAT, DIM), jnp.bfloat16),
        ],
        name="mem_prep",
    )(mem_key.T, mvp, mvp.T, out_w, perm)

    di = np.arange(DIM)
    hi = np.arange(128)
    ci = np.arange(CAT)
    sel = jnp.asarray((di[:, None] // HEAD_DIM == hi[None, :] % N_HEAD),
                      jnp.bfloat16)
    seg = jnp.asarray((ci[:, None] % N_HEAD == hi[None, :] % N_HEAD),
                      jnp.bfloat16)
    bias_pack = jnp.stack(
        [q_b, v_b, out_b, ln1_g, ln1_b, ln3_g, ln3_b, jnp.zeros_like(q_b)],
        axis=0)

    res = lambda shape: pl.BlockSpec(shape, lambda i: (0,) * len(shape))
    blk = lambda i: (i, 0)
    fp, ft, parts = pl.pallas_call(
        _main_kernel,
        grid=(G,),
        in_specs=[
            pl.BlockSpec((TB, DIM), blk),
            pl.BlockSpec((TB, DIM), blk),
            res((DIM, DIM)),
            res((DIM, DIM)),
            res((DIM, CAT)),
            res((CAT, DIM)),
            res((DIM, SLOT_PAD)),
            res((SLOT_PAD, DIM)),
            res((DIM, 128)),
            res((CAT, 128)),
            res((8, DIM)),
        ],
        out_specs=[
            pl.BlockSpec((TB, DIM), blk),
            pl.BlockSpec((TB, DIM), blk),
            pl.BlockSpec((1, 1, 128), lambda i: (i, 0, 0)),
        ],
        out_shape=[
            jax.ShapeDtypeStruct((N, DIM), f32),
            jax.ShapeDtypeStruct((N, DIM), f32),
            jax.ShapeDtypeStruct((G, 1, 128), f32),
        ],
        compiler_params=pltpu.CompilerParams(
            dimension_semantics=("parallel",),
            vmem_limit_bytes=48 * 1024 * 1024,
        ),
        name="mem_main",
    )(q2, v2, q_w.T.astype(jnp.bfloat16), v_w.T.astype(jnp.bfloat16),
      kpadt, wcat, vnpt, mvp.astype(jnp.bfloat16), sel, seg, bias_pack)

    f_predict = fp.reshape(B, S, C)
    f_target_recon = ft.reshape(B, S, C)
    recon_loss = jnp.sum(parts[:, 0, 0]) / N
    return (f_predict, f_target_recon, recon_loss, closs_arr[0, 0])


# speedup vs baseline: 1.2000x; 1.0287x over previous
import functools

import numpy as np

import jax
import jax.numpy as jnp
from jax.experimental import pallas as pl
from jax.experimental.pallas import tpu as pltpu

N_SLOT = 112
N_HEAD = 8
DIM = 512
HEAD_DIM = 64
SLOT_PAD = 128
CAT = N_HEAD * SLOT_PAD
RADIUS = 16.0
EPS = 1e-5
TB = 1024


def _f32dot(a, b):
    return jnp.dot(a, b, preferred_element_type=jnp.float32)


def _dot_rhs_t(a, b):
    return jax.lax.dot_general(a, b, (((1,), (1,)), ((), ())),
                               preferred_element_type=jnp.float32)


def _rownorm(x):
    ss = jnp.sum(x * x, axis=-1, keepdims=True)
    return x * jax.lax.rsqrt(jnp.maximum(ss, 1e-24))


def _ln(x, g, b):
    m = jnp.mean(x, axis=-1, keepdims=True)
    m2 = jnp.mean(x * x, axis=-1, keepdims=True)
    r = jax.lax.rsqrt(jnp.maximum(m2 - m * m, 0.0) + EPS)
    return (x - m) * r * g + b


def _prep_kernel(mkt_ref, mvp_ref, mvpt_ref, ow_ref, perm_ref,
                 kpadt_ref, wcat_ref, vnpt_ref, closs_ref,
                 kscr_ref, wscr_ref):
    mkt = mkt_ref[...]
    sst = jnp.sum(mkt * mkt, axis=0, keepdims=True)
    knt = mkt * jax.lax.rsqrt(jnp.maximum(sst, 1e-24))
    mvp = mvp_ref[...]
    for h in range(N_HEAD):
        bh = knt[:, h * N_SLOT:(h + 1) * N_SLOT]
        kscr_ref[h * HEAD_DIM:(h + 1) * HEAD_DIM, :] = jnp.pad(
            bh, ((0, 0), (h * SLOT_PAD, CAT - h * SLOT_PAD - N_SLOT)),
        ).astype(jnp.bfloat16)
        wscr_ref[h * SLOT_PAD:(h + 1) * SLOT_PAD, :] = _dot_rhs_t(
            mvp, ow_ref[:, h * DIM:(h + 1) * DIM]).astype(jnp.bfloat16)
    kpadt_ref[...] = _f32dot(kscr_ref[...], perm_ref[...]).astype(jnp.bfloat16)
    wcat_ref[...] = jax.lax.dot_general(
        perm_ref[...], wscr_ref[...], (((0,), (0,)), ((), ())),
        preferred_element_type=jnp.float32).astype(jnp.bfloat16)
    mvpt = mvpt_ref[...]
    ssv = jnp.sum(mvpt * mvpt, axis=0, keepdims=True)
    vnt = mvpt * jax.lax.rsqrt(jnp.maximum(ssv, 1e-24))
    vnpt_ref[...] = vnt.astype(jnp.bfloat16)
    gram = jax.lax.dot_general(vnt, vnt, (((0,), (0,)), ((), ())),
                               preferred_element_type=jnp.float32)
    r = jax.lax.broadcasted_iota(jnp.int32, (SLOT_PAD, SLOT_PAD), 0)
    c = jax.lax.broadcasted_iota(jnp.int32, (SLOT_PAD, SLOT_PAD), 1)
    eye = jnp.where((r == c) & (r < N_SLOT), 1.0, 0.0)
    closs_ref[...] = jnp.full((8, 128), jnp.sum(jnp.abs(eye - gram)) * 0.01,
                              jnp.float32)


def _main_kernel(q_ref, v_ref, qwt_ref, vwt_ref, kpadt_ref, wcat_ref,
                 vnpt_ref, mvp_ref, sel_ref, seg_ref,
                 bias_ref, fp_ref, ft_ref, part_ref):
    bf16 = jnp.bfloat16
    LOG2E = 1.4426950408889634
    ln1_g, ln1_b = bias_ref[3:4, :], bias_ref[4:5, :]

    def _half(r0, rows):
        sl = slice(r0, r0 + rows)
        q = q_ref[sl, :]
        v = v_ref[sl, :]

        qp = _f32dot(q.astype(bf16), qwt_ref[...]) + bias_ref[0:1, :]
        qpb = qp.astype(bf16)
        ssh = _f32dot(qpb * qpb, sel_ref[...])
        invc = jax.lax.rsqrt(jnp.maximum(ssh, 1e-24)) * (RADIUS * LOG2E)
        raw = _f32dot(qpb, kpadt_ref[...])
        e = jnp.exp2(raw * pltpu.repeat(invc, N_HEAD, axis=1))
        eb = e.astype(bf16)
        ssum = _f32dot(eb, seg_ref[...]) - 16.0
        rec = (1.0 / jnp.maximum(ssum, 1e-30)).astype(bf16)
        addr = eb * pltpu.repeat(rec, N_HEAD, axis=1)
        attn = _f32dot(addr, wcat_ref[...]) + bias_ref[2:3, :]
        fp_ref[sl, :] = _ln(q + attn, ln1_g, ln1_b)

        vp = _f32dot(v.astype(bf16), vwt_ref[...]) + bias_ref[1:2, :]
        ssv = jnp.sum(vp * vp, axis=-1, keepdims=True)
        rc = jax.lax.rsqrt(jnp.maximum(ssv, 1e-24)) * (RADIUS * LOG2E)
        raw2 = _f32dot(vp.astype(bf16), vnpt_ref[...])
        e2 = jnp.exp2(raw2 * rc)
        addr2 = e2 / (jnp.sum(e2, axis=-1, keepdims=True) - 16.0)
        ar = _f32dot(addr2.astype(bf16), mvp_ref[...])
        s_av = jnp.sum(ar * v, axis=-1, keepdims=True)
        s_aa = jnp.sum(ar * ar, axis=-1, keepdims=True)
        s_vv = jnp.sum(v * v, axis=-1, keepdims=True)
        cos = s_av * jax.lax.rsqrt(jnp.maximum(s_aa * s_vv, 1e-24))
        ft_ref[sl, :] = _ln(q + _ln(ar, bias_ref[5:6, :], bias_ref[6:7, :]),
                            ln1_g, ln1_b)
        return jnp.sum(jnp.abs(1.0 - cos))

    part = _half(0, TB)
    part_ref[...] = jnp.full((1, 1, 128), part, jnp.float32)


def kernel(query, value, mem_key, mem_value, q_w, q_b, v_w, v_b,
           out_w, out_b, ln1_g, ln1_b, ln3_g, ln3_b):
    B, S, C = query.shape
    N = B * S
    G = N // TB
    f32 = jnp.float32
    q2 = query.reshape(N, C)
    v2 = value.reshape(N, DIM)
    mvp = jnp.pad(mem_value, ((0, SLOT_PAD - N_SLOT), (0, 0)))

    co = np.arange(CAT)
    perm_np = np.zeros((CAT, CAT), np.float32)
    perm_np[co, (co % SLOT_PAD) * N_HEAD + co // SLOT_PAD] = 1.0
    perm = jnp.asarray(perm_np, jnp.bfloat16)

    kpadt, wcat, vnpt, closs_arr = pl.pallas_call(
        _prep_kernel,
        out_shape=[
            jax.ShapeDtypeStruct((DIM, CAT), jnp.bfloat16),
            jax.ShapeDtypeStruct((CAT, DIM), jnp.bfloat16),
            jax.ShapeDtypeStruct((DIM, SLOT_PAD), jnp.bfloat16),
            jax.ShapeDtypeStruct((8, 128), f32),
        ],
        scratch_shapes=[
            pltpu.VMEM((DIM, CAT), jnp.bfloat16),
            pltpu.VMEM((CAT, DIM), jnp.bfloat16),
        ],
        name="mem_prep",
    )(mem_key.T, mvp, mvp.T, out_w, perm)

    di = np.arange(DIM)
    hi = np.arange(128)
    ci = np.arange(CAT)
    sel = jnp.asarray((di[:, None] // HEAD_DIM == hi[None, :] % N_HEAD),
                      jnp.bfloat16)
    seg = jnp.asarray((ci[:, None] % N_HEAD == hi[None, :] % N_HEAD),
                      jnp.bfloat16)
    bias_pack = jnp.stack(
        [q_b, v_b, out_b, ln1_g, ln1_b, ln3_g, ln3_b, jnp.zeros_like(q_b)],
        axis=0)

    res = lambda shape: pl.BlockSpec(shape, lambda i: (0,) * len(shape))
    blk = lambda i: (i, 0)
    fp, ft, parts = pl.pallas_call(
        _main_kernel,
        grid=(G,),
        in_specs=[
            pl.BlockSpec((TB, DIM), blk),
            pl.BlockSpec((TB, DIM), blk),
            res((DIM, DIM)),
            res((DIM, DIM)),
            res((DIM, CAT)),
            res((CAT, DIM)),
            res((DIM, SLOT_PAD)),
            res((SLOT_PAD, DIM)),
            res((DIM, 128)),
            res((CAT, 128)),
            res((8, DIM)),
        ],
        out_specs=[
            pl.BlockSpec((TB, DIM), blk),
            pl.BlockSpec((TB, DIM), blk),
            pl.BlockSpec((1, 1, 128), lambda i: (i, 0, 0)),
        ],
        out_shape=[
            jax.ShapeDtypeStruct((N, DIM), f32),
            jax.ShapeDtypeStruct((N, DIM), f32),
            jax.ShapeDtypeStruct((G, 1, 128), f32),
        ],
        compiler_params=pltpu.CompilerParams(
            dimension_semantics=("parallel",),
            vmem_limit_bytes=56 * 1024 * 1024,
        ),
        name="mem_main",
    )(q2, v2, q_w.T.astype(jnp.bfloat16), v_w.T.astype(jnp.bfloat16),
      kpadt, wcat, vnpt, mvp.astype(jnp.bfloat16), sel, seg, bias_pack)

    f_predict = fp.reshape(B, S, C)
    f_target_recon = ft.reshape(B, S, C)
    recon_loss = jnp.sum(parts[:, 0, 0]) / N
    return (f_predict, f_target_recon, recon_loss, closs_arr[0, 0])


# speedup vs baseline: 1.2318x; 1.0265x over previous
import functools

import numpy as np

import jax
import jax.numpy as jnp
from jax.experimental import pallas as pl
from jax.experimental.pallas import tpu as pltpu

N_SLOT = 112
N_HEAD = 8
DIM = 512
HEAD_DIM = 64
SLOT_PAD = 128
CAT = N_HEAD * SLOT_PAD
RADIUS = 16.0
EPS = 1e-5
TB = 1024


def _f32dot(a, b):
    return jnp.dot(a, b, preferred_element_type=jnp.float32)


def _dot_rhs_t(a, b):
    return jax.lax.dot_general(a, b, (((1,), (1,)), ((), ())),
                               preferred_element_type=jnp.float32)


def _rownorm(x):
    ss = jnp.sum(x * x, axis=-1, keepdims=True)
    return x * jax.lax.rsqrt(jnp.maximum(ss, 1e-24))


def _ln(x, g, b):
    m = jnp.mean(x, axis=-1, keepdims=True)
    m2 = jnp.mean(x * x, axis=-1, keepdims=True)
    r = jax.lax.rsqrt(jnp.maximum(m2 - m * m, 0.0) + EPS)
    return (x - m) * r * g + b


def _prep_kernel(mk_ref, mv_ref, ow_ref, qw_ref, vw_ref, perm_ref, eye_ref,
                 qb_ref, vb_ref, ob_ref, l1g_ref, l1b_ref, l3g_ref, l3b_ref,
                 kpadt_ref, wcat_ref, vnpt_ref, qwt_ref, vwt_ref, mvpb_ref,
                 bias_ref, closs_ref, kscr_ref, wscr_ref):
    eye = eye_ref[...]
    qwt_ref[...] = _dot_rhs_t(eye, qw_ref[...]).astype(jnp.bfloat16)
    vwt_ref[...] = _dot_rhs_t(eye, vw_ref[...]).astype(jnp.bfloat16)
    bias_ref[0:1, :] = qb_ref[...]
    bias_ref[1:2, :] = vb_ref[...]
    bias_ref[2:3, :] = ob_ref[...]
    bias_ref[3:4, :] = l1g_ref[...]
    bias_ref[4:5, :] = l1b_ref[...]
    bias_ref[5:6, :] = l3g_ref[...]
    bias_ref[6:7, :] = l3b_ref[...]
    bias_ref[7:8, :] = jnp.zeros((1, DIM), jnp.float32)
    mkt = _dot_rhs_t(eye[:HEAD_DIM, :HEAD_DIM], mk_ref[...])
    sst = jnp.sum(mkt * mkt, axis=0, keepdims=True)
    knt = mkt * jax.lax.rsqrt(jnp.maximum(sst, 1e-24))
    mvp = jnp.concatenate(
        [mv_ref[...], jnp.zeros((SLOT_PAD - N_SLOT, DIM), jnp.float32)],
        axis=0)
    mvpb_ref[...] = mvp.astype(jnp.bfloat16)
    for h in range(N_HEAD):
        bh = knt[:, h * N_SLOT:(h + 1) * N_SLOT]
        kscr_ref[h * HEAD_DIM:(h + 1) * HEAD_DIM, :] = jnp.pad(
            bh, ((0, 0), (h * SLOT_PAD, CAT - h * SLOT_PAD - N_SLOT)),
        ).astype(jnp.bfloat16)
        wscr_ref[h * SLOT_PAD:(h + 1) * SLOT_PAD, :] = _dot_rhs_t(
            mvp, ow_ref[:, h * DIM:(h + 1) * DIM]).astype(jnp.bfloat16)
    kpadt_ref[...] = _f32dot(kscr_ref[...], perm_ref[...]).astype(jnp.bfloat16)
    wcat_ref[...] = jax.lax.dot_general(
        perm_ref[...], wscr_ref[...], (((0,), (0,)), ((), ())),
        preferred_element_type=jnp.float32).astype(jnp.bfloat16)
    mvpt = _dot_rhs_t(eye, mvp)
    ssv = jnp.sum(mvpt * mvpt, axis=0, keepdims=True)
    vnt = mvpt * jax.lax.rsqrt(jnp.maximum(ssv, 1e-24))
    vnpt_ref[...] = vnt.astype(jnp.bfloat16)
    gram = jax.lax.dot_general(vnt, vnt, (((0,), (0,)), ((), ())),
                               preferred_element_type=jnp.float32)
    r = jax.lax.broadcasted_iota(jnp.int32, (SLOT_PAD, SLOT_PAD), 0)
    c = jax.lax.broadcasted_iota(jnp.int32, (SLOT_PAD, SLOT_PAD), 1)
    eye = jnp.where((r == c) & (r < N_SLOT), 1.0, 0.0)
    closs_ref[...] = jnp.full((8, 128), jnp.sum(jnp.abs(eye - gram)) * 0.01,
                              jnp.float32)


def _main_kernel(q_ref, v_ref, qwt_ref, vwt_ref, kpadt_ref, wcat_ref,
                 vnpt_ref, mvp_ref, sel_ref, seg_ref,
                 bias_ref, fp_ref, ft_ref, part_ref):
    bf16 = jnp.bfloat16
    LOG2E = 1.4426950408889634
    ln1_g, ln1_b = bias_ref[3:4, :], bias_ref[4:5, :]

    def _half(r0, rows):
        sl = slice(r0, r0 + rows)
        q = q_ref[sl, :]
        v = v_ref[sl, :]

        qp = _f32dot(q.astype(bf16), qwt_ref[...]) + bias_ref[0:1, :]
        qpb = qp.astype(bf16)
        ssh = _f32dot(qpb * qpb, sel_ref[...])
        invc = jax.lax.rsqrt(jnp.maximum(ssh, 1e-24)) * (RADIUS * LOG2E)
        raw = _f32dot(qpb, kpadt_ref[...])
        e = jnp.exp2(raw * pltpu.repeat(invc, N_HEAD, axis=1))
        eb = e.astype(bf16)
        ssum = _f32dot(eb, seg_ref[...]) - 16.0
        rec = (1.0 / jnp.maximum(ssum, 1e-30)).astype(bf16)
        addr = eb * pltpu.repeat(rec, N_HEAD, axis=1)
        attn = _f32dot(addr, wcat_ref[...]) + bias_ref[2:3, :]
        fp_ref[sl, :] = _ln(q + attn, ln1_g, ln1_b)

        vp = _f32dot(v.astype(bf16), vwt_ref[...]) + bias_ref[1:2, :]
        ssv = jnp.sum(vp * vp, axis=-1, keepdims=True)
        rc = jax.lax.rsqrt(jnp.maximum(ssv, 1e-24)) * (RADIUS * LOG2E)
        raw2 = _f32dot(vp.astype(bf16), vnpt_ref[...])
        e2 = jnp.exp2(raw2 * rc)
        addr2 = e2 / (jnp.sum(e2, axis=-1, keepdims=True) - 16.0)
        ar = _f32dot(addr2.astype(bf16), mvp_ref[...])
        s_av = jnp.sum(ar * v, axis=-1, keepdims=True)
        s_aa = jnp.sum(ar * ar, axis=-1, keepdims=True)
        s_vv = jnp.sum(v * v, axis=-1, keepdims=True)
        cos = s_av * jax.lax.rsqrt(jnp.maximum(s_aa * s_vv, 1e-24))
        ft_ref[sl, :] = _ln(q + _ln(ar, bias_ref[5:6, :], bias_ref[6:7, :]),
                            ln1_g, ln1_b)
        return jnp.sum(jnp.abs(1.0 - cos))

    part = _half(0, TB)
    part_ref[...] = jnp.full((1, 1, 128), part, jnp.float32)


def kernel(query, value, mem_key, mem_value, q_w, q_b, v_w, v_b,
           out_w, out_b, ln1_g, ln1_b, ln3_g, ln3_b):
    B, S, C = query.shape
    N = B * S
    G = N // TB
    f32 = jnp.float32
    q2 = query.reshape(N, C)
    v2 = value.reshape(N, DIM)

    co = np.arange(CAT)
    perm_np = np.zeros((CAT, CAT), np.float32)
    perm_np[co, (co % SLOT_PAD) * N_HEAD + co // SLOT_PAD] = 1.0
    perm = jnp.asarray(perm_np, jnp.bfloat16)
    eye = jnp.asarray(np.eye(DIM, dtype=np.float32))

    bf = jnp.bfloat16
    r512 = lambda x: x.reshape(1, DIM)
    kpadt, wcat, vnpt, qwt, vwt, mvpb, bias_pack, closs_arr = pl.pallas_call(
        _prep_kernel,
        out_shape=[
            jax.ShapeDtypeStruct((DIM, CAT), bf),
            jax.ShapeDtypeStruct((CAT, DIM), bf),
            jax.ShapeDtypeStruct((DIM, SLOT_PAD), bf),
            jax.ShapeDtypeStruct((DIM, DIM), bf),
            jax.ShapeDtypeStruct((DIM, DIM), bf),
            jax.ShapeDtypeStruct((SLOT_PAD, DIM), bf),
            jax.ShapeDtypeStruct((8, DIM), f32),
            jax.ShapeDtypeStruct((8, 128), f32),
        ],
        scratch_shapes=[
            pltpu.VMEM((DIM, CAT), bf),
            pltpu.VMEM((CAT, DIM), bf),
        ],
        name="mem_prep",
    )(mem_key, mem_value, out_w, q_w, v_w, perm, eye,
      r512(q_b), r512(v_b), r512(out_b), r512(ln1_g), r512(ln1_b),
      r512(ln3_g), r512(ln3_b))

    di = np.arange(DIM)
    hi = np.arange(128)
    ci = np.arange(CAT)
    sel = jnp.asarray((di[:, None] // HEAD_DIM == hi[None, :] % N_HEAD),
                      jnp.bfloat16)
    seg = jnp.asarray((ci[:, None] % N_HEAD == hi[None, :] % N_HEAD),
                      jnp.bfloat16)

    res = lambda shape: pl.BlockSpec(shape, lambda i: (0,) * len(shape))
    blk = lambda i: (i, 0)
    fp, ft, parts = pl.pallas_call(
        _main_kernel,
        grid=(G,),
        in_specs=[
            pl.BlockSpec((TB, DIM), blk),
            pl.BlockSpec((TB, DIM), blk),
            res((DIM, DIM)),
            res((DIM, DIM)),
            res((DIM, CAT)),
            res((CAT, DIM)),
            res((DIM, SLOT_PAD)),
            res((SLOT_PAD, DIM)),
            res((DIM, 128)),
            res((CAT, 128)),
            res((8, DIM)),
        ],
        out_specs=[
            pl.BlockSpec((TB, DIM), blk),
            pl.BlockSpec((TB, DIM), blk),
            pl.BlockSpec((1, 1, 128), lambda i: (i, 0, 0)),
        ],
        out_shape=[
            jax.ShapeDtypeStruct((N, DIM), f32),
            jax.ShapeDtypeStruct((N, DIM), f32),
            jax.ShapeDtypeStruct((G, 1, 128), f32),
        ],
        compiler_params=pltpu.CompilerParams(
            dimension_semantics=("parallel",),
            vmem_limit_bytes=56 * 1024 * 1024,
        ),
        name="mem_main",
    )(q2, v2, qwt, vwt, kpadt, wcat, vnpt, mvpb, sel, seg, bias_pack)

    f_predict = fp.reshape(B, S, C)
    f_target_recon = ft.reshape(B, S, C)
    recon_loss = jnp.sum(parts[:, 0, 0]) / N
    return (f_predict, f_target_recon, recon_loss, closs_arr[0, 0])
